# baseline (speedup 1.0000x reference)
import functools

import jax
import jax.numpy as jnp
from jax import lax
from jax.experimental import pallas as pl
from jax.experimental.pallas import tpu as pltpu
from jax.experimental.pallas import tpu_sc as plsc

N = 10000
E = 320000
D = 128

NW = 32
RPW = 320
NPAD = NW * RPW
C = 64
IPTR_BUF = 336
IPTR_LEN = (NW - 1) * RPW + IPTR_BUF

_GATHER_DNUMS = lax.GatherDimensionNumbers(
    offset_dims=(), collapsed_slice_dims=(0,), start_index_map=(0,))


def _lane_gather(vec, idx):
    return lax.gather(vec, idx[:, None], _GATHER_DNUMS, (1,),
                      mode=lax.GatherScatterMode.PROMISE_IN_BOUNDS)


def _body(x_hbm, iptr_hbm, idx_hbm, val_hbm, out_hbm,
          iptr_v, idx_v, val_v, rows_v, acc_v,
          rsem0, rsem1, ivsem0, ivsem1):
    rsems = (rsem0, rsem1)
    ivsems = (ivsem0, ivsem1)
    cid = lax.axis_index("c")
    sid = lax.axis_index("s")
    w = sid * 2 + cid
    r0 = pl.multiple_of(w * RPW, RPW)

    pltpu.sync_copy(iptr_hbm.at[pl.ds(r0, IPTR_BUF)], iptr_v)

    zeros16 = jnp.zeros((16,), jnp.float32)

    def zero_body(i, carry):
        base = pl.multiple_of(i * 256, 256)
        for u in range(16):
            acc_v[pl.ds(base + u * 16, 16)] = zeros16
        return carry

    lax.fori_loop(0, RPW * D // 256, zero_body, 0)

    e_lo = iptr_v[pl.ds(0, 16)][0]
    e_hi = iptr_v[pl.ds(RPW, 16)][0]
    a0 = lax.bitwise_and(e_lo, jnp.int32(-16))
    nch = (e_hi - a0 + (C - 1)) // C
    iota16 = lax.iota(jnp.int32, 16)

    def chunk_start(k):
        return pl.multiple_of(jnp.minimum(a0 + k * C, E - C), 16)

    def chunk_slice(k):
        return pl.ds(chunk_start(k), C)

    def stage_iv(k, b):
        pltpu.async_copy(idx_hbm.at[chunk_slice(k)], idx_v.at[b], ivsems[b])
        pltpu.async_copy(val_hbm.at[chunk_slice(k)], val_v.at[b], ivsems[b])

    def wait_iv(k, b):
        pltpu.make_async_copy(idx_hbm.at[chunk_slice(k)], idx_v.at[b],
                              ivsems[b]).wait()
        pltpu.make_async_copy(val_hbm.at[chunk_slice(k)], val_v.at[b],
                              ivsems[b]).wait()

    H = C // 2

    def start_gather(b):
        for h in range(2):
            pltpu.async_copy(x_hbm.at[idx_v.at[b, pl.ds(h * H, H)]],
                             rows_v.at[b, pl.ds(h * H, H)], rsems[b])

    def wait_gather(b):
        for h in range(2):
            pltpu.make_async_copy(x_hbm.at[idx_v.at[b, pl.ds(h * H, H)]],
                                  rows_v.at[b, pl.ds(h * H, H)],
                                  rsems[b]).wait()

    @pl.when(nch > 0)
    def _prologue():
        stage_iv(0, 0)
        wait_iv(0, 0)
        start_gather(0)

        @pl.when(nch > 1)
        def _():
            stage_iv(1, 1)

    def process_chunk(k, b):
        s = chunk_start(k)
        lb = jnp.maximum(e_lo, a0 + k * C)
        nb = 1 - b

        @pl.when(k + 1 < nch)
        def _():
            wait_iv(k + 1, nb)
            start_gather(nb)

        wait_gather(b)

        groups = []
        for g in range(C // 16):
            pvec = s + g * 16 + iota16
            vblk = val_v[b, pl.ds(g * 16, 16)]
            in_range = (pvec >= lb) & (pvec < e_hi)
            vblk = jnp.where(in_range, vblk, 0.0)

            lo = jnp.zeros((16,), jnp.int32)
            hi = jnp.full((16,), RPW - 1, jnp.int32)
            for _ in range(9):
                mid = (lo + hi + 1) >> 1
                t = plsc.load_gather(iptr_v, [mid])
                pred = t <= pvec
                lo = jnp.where(pred, mid, lo)
                hi = jnp.where(pred, hi, mid - 1)
            groups.append((lo * D, vblk))

        @pl.when(k + 2 < nch)
        def _():
            stage_iv(k + 2, b)

        def emit_stores(ent):
            addr, prods = ent
            for d in range(8):
                plsc.addupdate(acc_v.at[pl.ds(addr + d * 16, 16)],
                               prods[d])

        for g in range(C // 16):
            rbase, vblk = groups[g]
            addrs = [rbase[l] for l in range(16)]

            def scaled(l):
                lsel = jnp.full((16,), l, jnp.int32)
                vvv = _lane_gather(vblk, lsel)
                j = g * 16 + l
                return [rows_v[b, j, pl.ds(d * 16, 16)] * vvv
                        for d in range(8)]

            one_row = addrs[0] == addrs[15]

            @pl.when(one_row)
            def _fast():
                sums = [scaled(l) for l in range(16)]
                stride = 1
                while stride < 16:
                    for l in range(0, 16, 2 * stride):
                        sums[l] = [a + c for a, c in
                                   zip(sums[l], sums[l + stride])]
                    stride *= 2
                emit_stores((addrs[0], sums[0]))

            @pl.when(jnp.logical_not(one_row))
            def _slow():
                pending = []
                for l in range(16):
                    prods = scaled(l)
                    if len(pending) == 2:
                        emit_stores(pending.pop(0))
                    pending.append((addrs[l], prods))
                for ent in pending:
                    emit_stores(ent)

    def pair_body(k2, carry):
        for b in (0, 1):
            k = 2 * k2 + b

            @pl.when(k < nch)
            def _():
                process_chunk(k, b)
        return carry

    lax.fori_loop(0, (nch + 1) // 2, pair_body, 0)

    pltpu.sync_copy(acc_v, out_hbm.at[pl.ds(pl.multiple_of(r0 * D, 16),
                                            RPW * D)])


@functools.partial(
    pl.kernel,
    out_type=jax.ShapeDtypeStruct((NPAD * D,), jnp.float32),
    mesh=plsc.VectorSubcoreMesh(core_axis_name="c", subcore_axis_name="s"),
    scratch_types=[
        pltpu.VMEM((IPTR_BUF,), jnp.int32),
        pltpu.VMEM((2, C), jnp.int32),
        pltpu.VMEM((2, C), jnp.float32),
        pltpu.VMEM((2, C, D), jnp.float32),
        pltpu.VMEM((RPW * D,), jnp.float32),
        pltpu.SemaphoreType.DMA,
        pltpu.SemaphoreType.DMA,
        pltpu.SemaphoreType.DMA,
        pltpu.SemaphoreType.DMA,
    ],
    compiler_params=pltpu.CompilerParams(needs_layout_passes=False),
)
def _sc_spmm(x_hbm, iptr_hbm, idx_hbm, val_hbm, out_hbm,
             iptr_v, idx_v, val_v, rows_v, acc_v,
             rsem0, rsem1, ivsem0, ivsem1):
    _body(x_hbm, iptr_hbm, idx_hbm, val_hbm, out_hbm,
          iptr_v, idx_v, val_v, rows_v, acc_v,
          rsem0, rsem1, ivsem0, ivsem1)


@jax.jit
def kernel(x, indptr, indices, values):
    iptr32 = indptr.astype(jnp.int32)
    iptr_pad = jnp.concatenate(
        [iptr32, jnp.full((IPTR_LEN - (N + 1),), E, jnp.int32)])
    out_flat = _sc_spmm(x, iptr_pad, indices.astype(jnp.int32), values)
    return out_flat.reshape(NPAD, D)[:N]


# speedup vs baseline: 1.8256x; 1.8256x over previous
import functools

import jax
import jax.numpy as jnp
from jax import lax
from jax.experimental import pallas as pl
from jax.experimental.pallas import tpu as pltpu
from jax.experimental.pallas import tpu_sc as plsc

N = 10000
E = 320000
D = 128

NW = 32
RPW = 320
NPAD = NW * RPW
C = 128
IPTR_BUF = 336
IPTR_LEN = (NW - 1) * RPW + IPTR_BUF

_GATHER_DNUMS = lax.GatherDimensionNumbers(
    offset_dims=(), collapsed_slice_dims=(0,), start_index_map=(0,))


def _lane_gather(vec, idx):
    return lax.gather(vec, idx[:, None], _GATHER_DNUMS, (1,),
                      mode=lax.GatherScatterMode.PROMISE_IN_BOUNDS)


def _body(x_hbm, iptr_hbm, idx_hbm, val_hbm, out_hbm,
          iptr_v, idx_v, val_v, rows_v, acc_v,
          rsem0, rsem1, ivsem0, ivsem1):
    rsems = (rsem0, rsem1)
    ivsems = (ivsem0, ivsem1)
    cid = lax.axis_index("c")
    sid = lax.axis_index("s")
    w = sid * 2 + cid
    r0 = pl.multiple_of(w * RPW, RPW)

    pltpu.sync_copy(iptr_hbm.at[pl.ds(r0, IPTR_BUF)], iptr_v)

    zeros16 = jnp.zeros((16,), jnp.float32)

    def zero_body(i, carry):
        base = pl.multiple_of(i * 256, 256)
        for u in range(16):
            acc_v[pl.ds(base + u * 16, 16)] = zeros16
        return carry

    lax.fori_loop(0, RPW * D // 256, zero_body, 0)

    e_lo = iptr_v[pl.ds(0, 16)][0]
    e_hi = iptr_v[pl.ds(RPW, 16)][0]
    a0 = lax.bitwise_and(e_lo, jnp.int32(-16))
    nch = (e_hi - a0 + (C - 1)) // C
    iota16 = lax.iota(jnp.int32, 16)

    def chunk_start(k):
        return pl.multiple_of(jnp.minimum(a0 + k * C, E - C), 16)

    def chunk_slice(k):
        return pl.ds(chunk_start(k), C)

    def stage_iv(k, b):
        pltpu.async_copy(idx_hbm.at[chunk_slice(k)], idx_v.at[b], ivsems[b])
        pltpu.async_copy(val_hbm.at[chunk_slice(k)], val_v.at[b], ivsems[b])

    def wait_iv(k, b):
        pltpu.make_async_copy(idx_hbm.at[chunk_slice(k)], idx_v.at[b],
                              ivsems[b]).wait()
        pltpu.make_async_copy(val_hbm.at[chunk_slice(k)], val_v.at[b],
                              ivsems[b]).wait()

    def start_gather(b):
        pltpu.async_copy(x_hbm.at[idx_v.at[b]], rows_v.at[b], rsems[b])

    def wait_gather(b):
        pltpu.make_async_copy(x_hbm.at[idx_v.at[b]], rows_v.at[b],
                              rsems[b]).wait()

    @pl.when(nch > 0)
    def _prologue():
        stage_iv(0, 0)
        wait_iv(0, 0)
        start_gather(0)

        @pl.when(nch > 1)
        def _():
            stage_iv(1, 1)

    def process_chunk(k, b):
        s = chunk_start(k)
        lb = jnp.maximum(e_lo, a0 + k * C)
        nb = 1 - b

        @pl.when(k + 1 < nch)
        def _():
            wait_iv(k + 1, nb)
            start_gather(nb)

        wait_gather(b)

        groups = []
        for g in range(C // 16):
            pvec = s + g * 16 + iota16
            vblk = val_v[b, pl.ds(g * 16, 16)]
            in_range = (pvec >= lb) & (pvec < e_hi)
            vblk = jnp.where(in_range, vblk, 0.0)

            lo = jnp.zeros((16,), jnp.int32)
            hi = jnp.full((16,), RPW - 1, jnp.int32)
            for _ in range(9):
                mid = (lo + hi + 1) >> 1
                t = plsc.load_gather(iptr_v, [mid])
                pred = t <= pvec
                lo = jnp.where(pred, mid, lo)
                hi = jnp.where(pred, hi, mid - 1)
            groups.append((lo * D, vblk))

        @pl.when(k + 2 < nch)
        def _():
            stage_iv(k + 2, b)

        def emit_stores(ent):
            addr, prods = ent
            for d in range(8):
                plsc.addupdate(acc_v.at[pl.ds(addr + d * 16, 16)],
                               prods[d])

        pending = []
        for g in range(C // 16):
            rbase, vblk = groups[g]
            addrs = [rbase[l] for l in range(16)]
            for l in range(16):
                j = g * 16 + l
                lsel = jnp.full((16,), l, jnp.int32)
                vvv = _lane_gather(vblk, lsel)
                prods = [rows_v[b, j, pl.ds(d * 16, 16)] * vvv
                         for d in range(8)]
                if len(pending) == 2:
                    emit_stores(pending.pop(0))
                pending.append((addrs[l], prods))
        for ent in pending:
            emit_stores(ent)

    def pair_body(k2, carry):
        for b in (0, 1):
            k = 2 * k2 + b

            @pl.when(k < nch)
            def _():
                process_chunk(k, b)
        return carry

    lax.fori_loop(0, (nch + 1) // 2, pair_body, 0)

    pltpu.sync_copy(acc_v, out_hbm.at[pl.ds(pl.multiple_of(r0 * D, 16),
                                            RPW * D)])


@functools.partial(
    pl.kernel,
    out_type=jax.ShapeDtypeStruct((NPAD * D,), jnp.float32),
    mesh=plsc.VectorSubcoreMesh(core_axis_name="c", subcore_axis_name="s"),
    scratch_types=[
        pltpu.VMEM((IPTR_BUF,), jnp.int32),
        pltpu.VMEM((2, C), jnp.int32),
        pltpu.VMEM((2, C), jnp.float32),
        pltpu.VMEM((2, C, D), jnp.float32),
        pltpu.VMEM((RPW * D,), jnp.float32),
        pltpu.SemaphoreType.DMA,
        pltpu.SemaphoreType.DMA,
        pltpu.SemaphoreType.DMA,
        pltpu.SemaphoreType.DMA,
    ],
    compiler_params=pltpu.CompilerParams(needs_layout_passes=False),
)
def _sc_spmm(x_hbm, iptr_hbm, idx_hbm, val_hbm, out_hbm,
             iptr_v, idx_v, val_v, rows_v, acc_v,
             rsem0, rsem1, ivsem0, ivsem1):
    _body(x_hbm, iptr_hbm, idx_hbm, val_hbm, out_hbm,
          iptr_v, idx_v, val_v, rows_v, acc_v,
          rsem0, rsem1, ivsem0, ivsem1)


@jax.jit
def kernel(x, indptr, indices, values):
    iptr32 = indptr.astype(jnp.int32)
    iptr_pad = jnp.concatenate(
        [iptr32, jnp.full((IPTR_LEN - (N + 1),), E, jnp.int32)])
    out_flat = _sc_spmm(x, iptr_pad, indices.astype(jnp.int32), values)
    return out_flat.reshape(NPAD, D)[:N]
